# baseline (device time: 73216 ns/iter reference)
import jax
import jax.numpy as jnp
from jax import lax
from jax.experimental import pallas as pl
from jax.experimental.pallas import tpu as pltpu

N_DEV = 8
PART_OFF = (0, 176, 344)
PART_ROWS = (176, 168, 168)


def kernel(x, w_mat, scale_x, scale_w):
    m_per, k = x.shape
    _, n_per = w_mat.shape

    def body(x_ref, w_ref, sx_ref, sw_ref, out_ref,
             xg0_ref, xg1_ref, xg2_ref, w8_ref,
             xs0_ref, xs1_ref, xs2_ref, w32_ref,
             send_sems, recv_sems, local_sems):
        xg = (xg0_ref, xg1_ref, xg2_ref)
        xs = (xs0_ref, xs1_ref, xs2_ref)
        my = lax.axis_index("i")

        q = my & 3
        my_y = q >> 1
        my_x = (q & 1) ^ my_y
        my_z = my >> 2
        nbr = (
            my ^ 1,
            (my & 4) | (3 - (my & 3)),
            my ^ 4,
        )
        my_slot = (
            my_x + 2 * my_y + 4 * my_z,
            my_y + 2 * my_z + 4 * my_x,
            my_z + 2 * my_x + 4 * my_y,
        )

        barrier = pltpu.get_barrier_semaphore()
        for a in range(3):
            pl.semaphore_signal(barrier, inc=1, device_id=(nbr[a],),
                                device_id_type=pl.DeviceIdType.MESH)

        xcopies = []
        for r in range(3):
            c = pltpu.make_async_copy(
                x_ref.at[pl.ds(PART_OFF[r], PART_ROWS[r])], xs[r],
                local_sems.at[r])
            c.start()
            xcopies.append(c)
        wcopy = pltpu.make_async_copy(w_ref, w32_ref, local_sems.at[3])
        wcopy.start()
        scale = sx_ref[0] * sw_ref[0]

        def band_gemm(r, slot):
            b0 = slot & 1
            b1 = (slot >> 1) & 1
            b2 = (slot >> 2) & 1
            ox, oy, oz = ((b0, b1, b2), (b2, b0, b1), (b1, b2, b0))[r]
            origin = 4 * oz + 2 * oy + (ox ^ oy)
            out_ref[pl.ds(origin * m_per + PART_OFF[r], PART_ROWS[r]), :] = (
                jnp.dot(xg[r][slot], w8_ref[...],
                        preferred_element_type=jnp.float32) * scale
            )

        def make_rdma(r, base, nslots, sem_idx, axis):
            return pltpu.make_async_remote_copy(
                src_ref=xg[r].at[pl.ds(base, nslots)],
                dst_ref=xg[r].at[pl.ds(base, nslots)],
                send_sem=send_sems.at[sem_idx],
                recv_sem=recv_sems.at[sem_idx],
                device_id=(nbr[axis],),
                device_id_type=pl.DeviceIdType.MESH,
            )

        def start_phase(p, half=None):
            rdmas = []
            for r in range(3):
                base = my_slot[r] & (N_DEV - (1 << p))
                if half is None:
                    rdmas.append(
                        make_rdma(r, base, 1 << p, 3 * p + r, (r + p) % 3))
                else:
                    rdmas.append(
                        make_rdma(r, base + 2 * half, 2, 6 + 3 * half + r,
                                  (r + p) % 3))
            for rdma in rdmas:
                rdma.start()
            return rdmas

        def recv_bands(p, half=None):
            for r in range(3):
                rb = (my_slot[r] & (N_DEV - (1 << p))) ^ (1 << p)
                lo, hi = (0, 1 << p) if half is None else (2 * half,
                                                           2 * half + 2)
                for i in range(lo, hi):
                    band_gemm(r, rb + i)

        pl.semaphore_wait(barrier, 3)
        ph0 = []
        for r in range(3):
            xcopies[r].wait()
            xg[r][my_slot[r]] = xs[r][...].astype(jnp.float8_e4m3fn)
            rdma = make_rdma(r, my_slot[r], 1, r, r)
            rdma.start()
            ph0.append(rdma)
        wcopy.wait()
        w8_ref[...] = w32_ref[...].astype(jnp.float8_e4m3fn)
        for r in range(3):
            band_gemm(r, my_slot[r])

        for rdma in ph0:
            rdma.wait()
        ph1 = start_phase(1)
        recv_bands(0)

        for rdma in ph1:
            rdma.wait()
        ph2a = start_phase(2, half=0)
        ph2b = start_phase(2, half=1)
        recv_bands(1)

        for rdma in ph2a:
            rdma.wait()
        recv_bands(2, half=0)
        for rdma in ph2b:
            rdma.wait()
        recv_bands(2, half=1)

    return pl.pallas_call(
        body,
        out_shape=jax.ShapeDtypeStruct((N_DEV * m_per, n_per), jnp.float32),
        in_specs=[
            pl.BlockSpec(memory_space=pltpu.MemorySpace.HBM),
            pl.BlockSpec(memory_space=pltpu.MemorySpace.HBM),
            pl.BlockSpec(memory_space=pltpu.SMEM),
            pl.BlockSpec(memory_space=pltpu.SMEM),
        ],
        out_specs=pl.BlockSpec(memory_space=pltpu.VMEM),
        scratch_shapes=[
            pltpu.VMEM((N_DEV, PART_ROWS[0], k), jnp.float8_e4m3fn),
            pltpu.VMEM((N_DEV, PART_ROWS[1], k), jnp.float8_e4m3fn),
            pltpu.VMEM((N_DEV, PART_ROWS[2], k), jnp.float8_e4m3fn),
            pltpu.VMEM((k, n_per), jnp.float8_e4m3fn),
            pltpu.VMEM((PART_ROWS[0], k), jnp.float32),
            pltpu.VMEM((PART_ROWS[1], k), jnp.float32),
            pltpu.VMEM((PART_ROWS[2], k), jnp.float32),
            pltpu.VMEM((k, n_per), jnp.float32),
            pltpu.SemaphoreType.DMA((12,)),
            pltpu.SemaphoreType.DMA((12,)),
            pltpu.SemaphoreType.DMA((4,)),
        ],
        compiler_params=pltpu.CompilerParams(collective_id=0),
    )(x, w_mat, scale_x, scale_w)


# device time: 68858 ns/iter; 1.0633x vs baseline; 1.0633x over previous
import jax
import jax.numpy as jnp
from jax import lax
from jax.experimental import pallas as pl
from jax.experimental.pallas import tpu as pltpu

N_DEV = 8
PART_OFF = (0, 176, 344)
PART_ROWS = (176, 168, 168)


def kernel(x, w_mat, scale_x, scale_w):
    m_per, k = x.shape
    _, n_per = w_mat.shape

    def body(x_ref, w_ref, sx_ref, sw_ref, out_ref,
             xg0_ref, xg1_ref, xg2_ref, w8_ref,
             xs0_ref, xs1_ref, xs2_ref, w32_ref,
             send_sems, recv_sems, local_sems):
        xg = (xg0_ref, xg1_ref, xg2_ref)
        xs = (xs0_ref, xs1_ref, xs2_ref)
        my = lax.axis_index("i")

        q = my & 3
        my_y = q >> 1
        my_x = (q & 1) ^ my_y
        my_z = my >> 2
        nbr = (
            my ^ 1,
            (my & 4) | (3 - (my & 3)),
            my ^ 4,
        )
        my_slot = (
            my_x + 2 * my_y + 4 * my_z,
            my_y + 2 * my_z + 4 * my_x,
            my_z + 2 * my_x + 4 * my_y,
        )

        barrier = pltpu.get_barrier_semaphore()
        for a in range(3):
            pl.semaphore_signal(barrier, inc=1, device_id=(nbr[a],),
                                device_id_type=pl.DeviceIdType.MESH)

        xcopies = []
        for r in range(3):
            c = pltpu.make_async_copy(
                x_ref.at[pl.ds(PART_OFF[r], PART_ROWS[r])], xs[r],
                local_sems.at[r])
            c.start()
            xcopies.append(c)
        wcopy = pltpu.make_async_copy(w_ref, w32_ref, local_sems.at[3])
        wcopy.start()
        scale = sx_ref[0] * sw_ref[0]

        def band_gemm(r, slot):
            b0 = slot & 1
            b1 = (slot >> 1) & 1
            b2 = (slot >> 2) & 1
            ox, oy, oz = ((b0, b1, b2), (b2, b0, b1), (b1, b2, b0))[r]
            origin = 4 * oz + 2 * oy + (ox ^ oy)
            out_ref[pl.ds(origin * m_per + PART_OFF[r], PART_ROWS[r]), :] = (
                jnp.dot(xg[r][slot], w8_ref[...],
                        preferred_element_type=jnp.float32) * scale
            )

        def rdma_k(r, slot, axis, kk):
            return pltpu.make_async_remote_copy(
                src_ref=xg[r].at[slot],
                dst_ref=xg[r].at[slot],
                send_sem=send_sems.at[7 * r + kk],
                recv_sem=recv_sems.at[7 * r + kk],
                device_id=(nbr[axis],),
                device_id_type=pl.DeviceIdType.MESH,
            )

        sends = []

        def fwd(r, slot, axis, kk):
            d = rdma_k(r, slot, axis, kk)
            d.start()
            sends.append(d)

        pl.semaphore_wait(barrier, 3)
        for r in range(3):
            xcopies[r].wait()
            xg[r][my_slot[r]] = xs[r][...].astype(jnp.float8_e4m3fn)
            fwd(r, my_slot[r], r, 0)
        for r in range(3):
            fwd(r, my_slot[r], (r + 1) % 3, 1)
        for r in range(3):
            fwd(r, my_slot[r], (r + 2) % 3, 2)

        wcopy.wait()
        w8_ref[...] = w32_ref[...].astype(jnp.float8_e4m3fn)
        for r in range(3):
            band_gemm(r, my_slot[r])

        for r in range(3):
            rdma_k(r, my_slot[r] ^ 1, r, 0).wait_recv()
            fwd(r, my_slot[r] ^ 1, (r + 1) % 3, 3)
            fwd(r, my_slot[r] ^ 1, (r + 2) % 3, 4)
        for r in range(3):
            band_gemm(r, my_slot[r] ^ 1)

        for r in range(3):
            rdma_k(r, my_slot[r] ^ 2, (r + 1) % 3, 1).wait_recv()
            fwd(r, my_slot[r] ^ 2, (r + 2) % 3, 5)
        for r in range(3):
            band_gemm(r, my_slot[r] ^ 2)
        for r in range(3):
            rdma_k(r, my_slot[r] ^ 3, (r + 1) % 3, 3).wait_recv()
            fwd(r, my_slot[r] ^ 3, (r + 2) % 3, 6)
        for r in range(3):
            band_gemm(r, my_slot[r] ^ 3)

        for kk, flip in ((2, 4), (4, 5), (5, 6), (6, 7)):
            for r in range(3):
                rdma_k(r, my_slot[r] ^ flip, (r + 2) % 3, kk).wait_recv()
                band_gemm(r, my_slot[r] ^ flip)

        for d in sends:
            d.wait_send()

    return pl.pallas_call(
        body,
        out_shape=jax.ShapeDtypeStruct((N_DEV * m_per, n_per), jnp.float32),
        in_specs=[
            pl.BlockSpec(memory_space=pltpu.MemorySpace.HBM),
            pl.BlockSpec(memory_space=pltpu.MemorySpace.HBM),
            pl.BlockSpec(memory_space=pltpu.SMEM),
            pl.BlockSpec(memory_space=pltpu.SMEM),
        ],
        out_specs=pl.BlockSpec(memory_space=pltpu.VMEM),
        scratch_shapes=[
            pltpu.VMEM((N_DEV, PART_ROWS[0], k), jnp.float8_e4m3fn),
            pltpu.VMEM((N_DEV, PART_ROWS[1], k), jnp.float8_e4m3fn),
            pltpu.VMEM((N_DEV, PART_ROWS[2], k), jnp.float8_e4m3fn),
            pltpu.VMEM((k, n_per), jnp.float8_e4m3fn),
            pltpu.VMEM((PART_ROWS[0], k), jnp.float32),
            pltpu.VMEM((PART_ROWS[1], k), jnp.float32),
            pltpu.VMEM((PART_ROWS[2], k), jnp.float32),
            pltpu.VMEM((k, n_per), jnp.float32),
            pltpu.SemaphoreType.DMA((21,)),
            pltpu.SemaphoreType.DMA((21,)),
            pltpu.SemaphoreType.DMA((4,)),
        ],
        compiler_params=pltpu.CompilerParams(collective_id=0),
    )(x, w_mat, scale_x, scale_w)
